# Initial kernel scaffold; baseline (speedup 1.0000x reference)
import jax
import jax.numpy as jnp
from jax.experimental import pallas as pl


def kernel(x, edge_index, W1, b1, W2, b2):
    raise NotImplementedError("write your pallas kernel here")


# speedup vs baseline: 9.7244x; 9.7244x over previous
import functools

import jax
import jax.numpy as jnp
from jax import lax
from jax.experimental import pallas as pl
from jax.experimental.pallas import tpu as pltpu
from jax.experimental.pallas import tpu_sc as plsc

N = 10000
E = 320000
D = 128

NC = 2
NS = 16
NW = NC * NS
L = 16

N_PAD = 10240
SLAB = N_PAD // NS
K = 128
CPW = 80
EW = CPW * K
E_PAD = EW * NW
DUMMY = N

_MESH = plsc.VectorSubcoreMesh(core_axis_name="c", subcore_axis_name="s")


def _deg_body(dst_hbm, out_hbm, dacc, idx_d, ones_v, zrow):
    cid = lax.axis_index("c")
    sid = lax.axis_index("s")
    wid = sid * NC + cid

    def _zfill(i, carry):
        zrow[pl.ds(i * L, L)] = jnp.zeros((L,), jnp.float32)
        return carry

    lax.fori_loop(0, SLAB // L, _zfill, 0)
    for j in range(K // L):
        ones_v[pl.ds(j * L, L)] = jnp.ones((L,), jnp.float32)

    pltpu.sync_copy(zrow, dacc.at[pl.ds(sid * SLAB, SLAB)])
    pltpu.sync_copy(dst_hbm.at[wid], idx_d)
    plsc.subcore_barrier()

    def _chunk(c, carry):
        pltpu.sync_copy(ones_v, dacc.at[idx_d.at[c]], add=True)
        return carry

    lax.fori_loop(0, CPW, _chunk, 0)
    plsc.subcore_barrier()
    pltpu.sync_copy(dacc.at[pl.ds(sid * SLAB, SLAB)],
                    out_hbm.at[cid, pl.ds(sid * SLAB, SLAB)])


_deg = pl.kernel(
    _deg_body,
    out_type=jax.ShapeDtypeStruct((NC, N_PAD), jnp.float32),
    mesh=_MESH,
    scratch_types=[
        pltpu.VMEM_SHARED((N_PAD,), jnp.float32),
        pltpu.VMEM((CPW, K), jnp.int32),
        pltpu.VMEM((K,), jnp.float32),
        pltpu.VMEM((SLAB,), jnp.float32),
    ],
)


SB = 16
NSB = CPW // SB


def _agg_body(g_hbm, src_hbm, dst_hbm, out_hbm,
              acc, idx_s, idx_d, rows, zbuf, sem0, sem1):
    cid = lax.axis_index("c")
    sid = lax.axis_index("s")
    wid = sid * NC + cid

    def _zfill(i, carry):
        for j in range(D // L):
            zbuf[i, pl.ds(j * L, L)] = jnp.zeros((L,), jnp.float32)
        return carry

    lax.fori_loop(0, 8, _zfill, 0)

    def _zslab(c, carry):
        pltpu.sync_copy(zbuf, acc.at[pl.ds(sid * SLAB + c * 8, 8)])
        return carry

    lax.fori_loop(0, SLAB // 8, _zslab, 0)
    plsc.subcore_barrier()

    sems = (sem0, sem1)

    def _superblock(sb, carry):
        pltpu.sync_copy(src_hbm.at[wid, pl.ds(sb * SB, SB)], idx_s)
        pltpu.sync_copy(dst_hbm.at[wid, pl.ds(sb * SB, SB)], idx_d)
        pltpu.async_copy(g_hbm.at[idx_s.at[0]], rows.at[0], sem0)

        def _chunks(i, carry2):
            for b in range(2):
                c = i * 2 + b
                nxt = c + 1

                @pl.when(nxt < SB)
                def _start():
                    pltpu.async_copy(g_hbm.at[idx_s.at[nxt]], rows.at[1 - b],
                                     sems[1 - b])

                pltpu.make_async_copy(g_hbm.at[idx_s.at[c]], rows.at[b],
                                      sems[b]).wait()
                pltpu.sync_copy(rows.at[b], acc.at[idx_d.at[c]], add=True)
            return carry2

        lax.fori_loop(0, SB // 2, _chunks, 0)
        return carry

    lax.fori_loop(0, NSB, _superblock, 0)
    plsc.subcore_barrier()
    pltpu.sync_copy(acc.at[pl.ds(sid * SLAB, SLAB)],
                    out_hbm.at[cid, pl.ds(sid * SLAB, SLAB)])


_agg = pl.kernel(
    _agg_body,
    out_type=jax.ShapeDtypeStruct((NC, N_PAD, D), jnp.float32),
    mesh=_MESH,
    scratch_types=[
        pltpu.VMEM_SHARED((N_PAD, D), jnp.float32),
        pltpu.VMEM((SB, K), jnp.int32),
        pltpu.VMEM((SB, K), jnp.int32),
        pltpu.VMEM((2, K, D), jnp.float32),
        pltpu.VMEM((8, D), jnp.float32),
        pltpu.SemaphoreType.DMA,
        pltpu.SemaphoreType.DMA,
    ],
)


def _lin1_body(x_ref, w_ref, deg_ref, g_ref, dis_ref):
    deg = deg_ref[0] + deg_ref[1] + 1.0
    dis = lax.rsqrt(deg)
    h = jnp.dot(x_ref[...], w_ref[...], preferred_element_type=jnp.float32)
    g_ref[...] = h * dis
    dis_ref[...] = dis


_lin1 = pl.pallas_call(
    _lin1_body,
    out_shape=(jax.ShapeDtypeStruct((N_PAD, D), jnp.float32),
               jax.ShapeDtypeStruct((N_PAD, 1), jnp.float32)),
)


def _lin2_body(a_ref, g_ref, dis_ref, b_ref, w_ref, g2_ref):
    s = a_ref[0] + a_ref[1] + g_ref[...]
    z = jnp.maximum(s * dis_ref[...] + b_ref[...], 0.0)
    g2_ref[...] = jnp.dot(z, w_ref[...],
                          preferred_element_type=jnp.float32) * dis_ref[...]


_lin2 = pl.pallas_call(
    _lin2_body,
    out_shape=jax.ShapeDtypeStruct((N_PAD, D), jnp.float32),
)


def _out_body(a_ref, g_ref, dis_ref, b_ref, o_ref):
    o_ref[...] = (a_ref[0] + a_ref[1] + g_ref[...]) * dis_ref[...] + b_ref[...]


_out = pl.pallas_call(
    _out_body,
    out_shape=jax.ShapeDtypeStruct((N_PAD, D), jnp.float32),
)


def kernel(x, edge_index, W1, b1, W2, b2):
    pad = jnp.full((E_PAD - E,), DUMMY, jnp.int32)
    src = jnp.concatenate([edge_index[0], pad]).reshape(NW, CPW, K)
    dst = jnp.concatenate([edge_index[1], pad]).reshape(NW, CPW, K)
    x_pad = jnp.zeros((N_PAD, D), jnp.float32).at[:N].set(x)

    deg2 = _deg(dst).reshape(NC, N_PAD, 1)
    g1, dis = _lin1(x_pad, W1, deg2)
    A1 = _agg(g1, src, dst)
    g2 = _lin2(A1, g1, dis, b1.reshape(1, D), W2)
    A2 = _agg(g2, src, dst)
    out = _out(A2, g2, dis, b2.reshape(1, D))
    return out[:N]


# speedup vs baseline: 29.9657x; 3.0815x over previous
import functools

import jax
import jax.numpy as jnp
from jax import lax
from jax.experimental import pallas as pl
from jax.experimental.pallas import tpu as pltpu
from jax.experimental.pallas import tpu_sc as plsc

N = 10000
E = 320000
D = 128

NC = 2
NS = 16
NW = NC * NS
L = 16

N_PAD = 10240
SLAB = N_PAD // NS
K = 128
CPW = 80
EW = CPW * K
E_PAD = EW * NW
DUMMY = N

_MESH = plsc.VectorSubcoreMesh(core_axis_name="c", subcore_axis_name="s")


def _deg_body(dst_hbm, out_hbm, dacc, idx_d, ones_v, zrow):
    cid = lax.axis_index("c")
    sid = lax.axis_index("s")
    wid = sid * NC + cid

    def _zfill(i, carry):
        zrow[pl.ds(i * L, L)] = jnp.zeros((L,), jnp.float32)
        return carry

    lax.fori_loop(0, SLAB // L, _zfill, 0)
    for j in range(K // L):
        ones_v[pl.ds(j * L, L)] = jnp.ones((L,), jnp.float32)

    pltpu.sync_copy(zrow, dacc.at[pl.ds(sid * SLAB, SLAB)])
    pltpu.sync_copy(dst_hbm.at[wid], idx_d)
    plsc.subcore_barrier()

    def _chunk(c, carry):
        pltpu.sync_copy(ones_v, dacc.at[idx_d.at[c]], add=True)
        return carry

    lax.fori_loop(0, CPW, _chunk, 0)
    plsc.subcore_barrier()
    pltpu.sync_copy(dacc.at[pl.ds(sid * SLAB, SLAB)],
                    out_hbm.at[cid, pl.ds(sid * SLAB, SLAB)])


_deg = pl.kernel(
    _deg_body,
    out_type=jax.ShapeDtypeStruct((NC, N_PAD), jnp.float32),
    mesh=_MESH,
    scratch_types=[
        pltpu.VMEM_SHARED((N_PAD,), jnp.float32),
        pltpu.VMEM((CPW, K), jnp.int32),
        pltpu.VMEM((K,), jnp.float32),
        pltpu.VMEM((SLAB,), jnp.float32),
    ],
)


SB = 16
NSB = CPW // SB


def _agg_body(g_hbm, src_hbm, dst_hbm, out_hbm,
              acc, idx_s, idx_d, rows, zbuf, sem0, sem1):
    cid = lax.axis_index("c")
    sid = lax.axis_index("s")
    wid = sid * NC + cid

    def _zfill(i, carry):
        for j in range(D // L):
            zbuf[i, pl.ds(j * L, L)] = jnp.zeros((L,), jnp.float32)
        return carry

    lax.fori_loop(0, 8, _zfill, 0)

    def _zslab(c, carry):
        pltpu.sync_copy(zbuf, acc.at[pl.ds(sid * SLAB + c * 8, 8)])
        return carry

    lax.fori_loop(0, SLAB // 8, _zslab, 0)
    plsc.subcore_barrier()

    sems = (sem0, sem1)

    def _superblock(sb, carry):
        pltpu.sync_copy(src_hbm.at[wid, pl.ds(sb * SB, SB)], idx_s)
        pltpu.sync_copy(dst_hbm.at[wid, pl.ds(sb * SB, SB)], idx_d)
        pltpu.async_copy(g_hbm.at[idx_s.at[0]], rows.at[0], sem0)

        def _chunks(i, carry2):
            for b in range(2):
                c = i * 2 + b
                nxt = c + 1

                @pl.when(nxt < SB)
                def _start():
                    pltpu.async_copy(g_hbm.at[idx_s.at[nxt]], rows.at[1 - b],
                                     sems[1 - b])

                pltpu.make_async_copy(g_hbm.at[idx_s.at[c]], rows.at[b],
                                      sems[b]).wait()
                pltpu.sync_copy(rows.at[b], acc.at[idx_d.at[c]], add=True)
            return carry2

        lax.fori_loop(0, SB // 2, _chunks, 0)
        return carry

    lax.fori_loop(0, NSB, _superblock, 0)
    plsc.subcore_barrier()
    pltpu.sync_copy(acc.at[pl.ds(sid * SLAB, SLAB)],
                    out_hbm.at[cid, pl.ds(sid * SLAB, SLAB)])


_agg = pl.kernel(
    _agg_body,
    out_type=jax.ShapeDtypeStruct((NC, N_PAD, D), jnp.float32),
    mesh=_MESH,
    scratch_types=[
        pltpu.VMEM_SHARED((N_PAD, D), jnp.float32),
        pltpu.VMEM((SB, K), jnp.int32),
        pltpu.VMEM((SB, K), jnp.int32),
        pltpu.VMEM((2, K, D), jnp.float32),
        pltpu.VMEM((8, D), jnp.float32),
        pltpu.SemaphoreType.DMA,
        pltpu.SemaphoreType.DMA,
    ],
)


def _lin1_body(x_ref, w_ref, deg_ref, g_ref, dis_ref):
    deg = deg_ref[0] + deg_ref[1] + 1.0
    dis = lax.rsqrt(deg)
    h = jnp.dot(x_ref[...], w_ref[...], preferred_element_type=jnp.float32)
    g_ref[...] = h * dis
    dis_ref[...] = dis


_lin1 = pl.pallas_call(
    _lin1_body,
    out_shape=(jax.ShapeDtypeStruct((N_PAD, D), jnp.float32),
               jax.ShapeDtypeStruct((N_PAD, 1), jnp.float32)),
)


def _lin2_body(a_ref, g_ref, dis_ref, b_ref, w_ref, g2_ref):
    s = a_ref[0] + a_ref[1] + g_ref[...]
    z = jnp.maximum(s * dis_ref[...] + b_ref[...], 0.0)
    g2_ref[...] = jnp.dot(z, w_ref[...],
                          preferred_element_type=jnp.float32) * dis_ref[...]


_lin2 = pl.pallas_call(
    _lin2_body,
    out_shape=jax.ShapeDtypeStruct((N_PAD, D), jnp.float32),
)


def _out_body(a_ref, g_ref, dis_ref, b_ref, o_ref):
    o_ref[...] = (a_ref[0] + a_ref[1] + g_ref[...]) * dis_ref[...] + b_ref[...]


_out = pl.pallas_call(
    _out_body,
    out_shape=jax.ShapeDtypeStruct((N_PAD, D), jnp.float32),
)


def _shard(a):
    return a.reshape(EW, NW).T.reshape(NW, CPW, K)


def kernel(x, edge_index, W1, b1, W2, b2):
    pad = DUMMY + (jnp.arange(E_PAD - E, dtype=jnp.int32) % (N_PAD - N - 1))
    src = _shard(jnp.concatenate([edge_index[0], pad]))
    dst = _shard(jnp.concatenate([edge_index[1], pad]))
    x_pad = jnp.zeros((N_PAD, D), jnp.float32).at[:N].set(x)

    deg2 = _deg(dst).reshape(NC, N_PAD, 1)
    g1, dis = _lin1(x_pad, W1, deg2)
    A1 = _agg(g1, src, dst)
    g2 = _lin2(A1, g1, dis, b1.reshape(1, D), W2)
    A2 = _agg(g2, src, dst)
    out = _out(A2, g2, dis, b2.reshape(1, D))
    return out[:N]


# speedup vs baseline: 32.8373x; 1.0958x over previous
import functools

import jax
import jax.numpy as jnp
from jax import lax
from jax.experimental import pallas as pl
from jax.experimental.pallas import tpu as pltpu
from jax.experimental.pallas import tpu_sc as plsc

N = 10000
E = 320000
D = 128

NC = 2
NS = 16
NW = NC * NS
L = 16

N_PAD = 10240
SLAB = N_PAD // NS
K = 128
CPW = 80
EW = CPW * K
E_PAD = EW * NW
DUMMY = N

_MESH = plsc.VectorSubcoreMesh(core_axis_name="c", subcore_axis_name="s")


def _deg_body(dst_hbm, out_hbm, dacc, idx_d, ones_v, zrow):
    cid = lax.axis_index("c")
    sid = lax.axis_index("s")
    wid = sid * NC + cid

    def _zfill(i, carry):
        zrow[pl.ds(i * L, L)] = jnp.zeros((L,), jnp.float32)
        return carry

    lax.fori_loop(0, SLAB // L, _zfill, 0)
    for j in range(K // L):
        ones_v[pl.ds(j * L, L)] = jnp.ones((L,), jnp.float32)

    pltpu.sync_copy(zrow, dacc.at[pl.ds(sid * SLAB, SLAB)])
    pltpu.sync_copy(dst_hbm.at[wid], idx_d)
    plsc.subcore_barrier()

    def _chunk(c, carry):
        pltpu.sync_copy(ones_v, dacc.at[idx_d.at[c]], add=True)
        return carry

    lax.fori_loop(0, CPW, _chunk, 0)
    plsc.subcore_barrier()
    pltpu.sync_copy(dacc.at[pl.ds(sid * SLAB, SLAB)],
                    out_hbm.at[cid, pl.ds(sid * SLAB, SLAB)])


_deg = pl.kernel(
    _deg_body,
    out_type=jax.ShapeDtypeStruct((NC, N_PAD), jnp.float32),
    mesh=_MESH,
    scratch_types=[
        pltpu.VMEM_SHARED((N_PAD,), jnp.float32),
        pltpu.VMEM((CPW, K), jnp.int32),
        pltpu.VMEM((K,), jnp.float32),
        pltpu.VMEM((SLAB,), jnp.float32),
    ],
)


SB = 8
NSB = CPW // SB


def _agg_body(g_hbm, src_hbm, dst_hbm, out_hbm,
              acc, idx_s, idx_d, rows, zbuf, sem0, sem1, semi0, semi1):
    cid = lax.axis_index("c")
    sid = lax.axis_index("s")
    wid = sid * NC + cid

    sems = (sem0, sem1)
    semi = (semi0, semi1)

    def _start_idx(sb, b):
        pltpu.async_copy(src_hbm.at[wid, pl.ds(sb * SB, SB)], idx_s.at[b],
                         semi[b])
        pltpu.async_copy(dst_hbm.at[wid, pl.ds(sb * SB, SB)], idx_d.at[b],
                         semi[b])

    def _wait_idx(sb, b):
        pltpu.make_async_copy(src_hbm.at[wid, pl.ds(sb * SB, SB)],
                              idx_s.at[b], semi[b]).wait()
        pltpu.make_async_copy(dst_hbm.at[wid, pl.ds(sb * SB, SB)],
                              idx_d.at[b], semi[b]).wait()

    _start_idx(0, 0)

    def _zfill(i, carry):
        for j in range(D // L):
            zbuf[i, pl.ds(j * L, L)] = jnp.zeros((L,), jnp.float32)
        return carry

    lax.fori_loop(0, 8, _zfill, 0)

    def _zslab(c, carry):
        pltpu.sync_copy(zbuf, acc.at[pl.ds(sid * SLAB + c * 8, 8)])
        return carry

    lax.fori_loop(0, SLAB // 8, _zslab, 0)

    _wait_idx(0, 0)
    pltpu.async_copy(g_hbm.at[idx_s.at[0, 0]], rows.at[0], sem0)
    plsc.subcore_barrier()

    def _pair(pair, carry):
        for p in range(2):
            sb = pair * 2 + p

            @pl.when(sb + 1 < NSB)
            def _prefetch():
                _start_idx(sb + 1, 1 - p)

            def _chunks(i, carry2):
                for b in range(2):
                    c = i * 2 + b
                    pltpu.async_copy(g_hbm.at[idx_s.at[p, c + 1]],
                                     rows.at[1 - b], sems[1 - b])
                    pltpu.make_async_copy(g_hbm.at[idx_s.at[p, c]],
                                          rows.at[b], sems[b]).wait()
                    pltpu.sync_copy(rows.at[b], acc.at[idx_d.at[p, c]],
                                    add=True)
                return carry2

            lax.fori_loop(0, SB // 2 - 1, _chunks, 0)

            pltpu.async_copy(g_hbm.at[idx_s.at[p, SB - 1]], rows.at[1],
                             sems[1])
            pltpu.make_async_copy(g_hbm.at[idx_s.at[p, SB - 2]], rows.at[0],
                                  sems[0]).wait()
            pltpu.sync_copy(rows.at[0], acc.at[idx_d.at[p, SB - 2]], add=True)

            @pl.when(sb + 1 < NSB)
            def _prime_next():
                _wait_idx(sb + 1, 1 - p)
                pltpu.async_copy(g_hbm.at[idx_s.at[1 - p, 0]], rows.at[0],
                                 sems[0])

            pltpu.make_async_copy(g_hbm.at[idx_s.at[p, SB - 1]], rows.at[1],
                                  sems[1]).wait()
            pltpu.sync_copy(rows.at[1], acc.at[idx_d.at[p, SB - 1]], add=True)
        return carry

    lax.fori_loop(0, NSB // 2, _pair, 0)
    plsc.subcore_barrier()
    pltpu.sync_copy(acc.at[pl.ds(sid * SLAB, SLAB)],
                    out_hbm.at[cid, pl.ds(sid * SLAB, SLAB)])


_agg = pl.kernel(
    _agg_body,
    out_type=jax.ShapeDtypeStruct((NC, N_PAD, D), jnp.float32),
    mesh=_MESH,
    scratch_types=[
        pltpu.VMEM_SHARED((N_PAD, D), jnp.float32),
        pltpu.VMEM((2, SB, K), jnp.int32),
        pltpu.VMEM((2, SB, K), jnp.int32),
        pltpu.VMEM((2, K, D), jnp.float32),
        pltpu.VMEM((8, D), jnp.float32),
        pltpu.SemaphoreType.DMA,
        pltpu.SemaphoreType.DMA,
        pltpu.SemaphoreType.DMA,
        pltpu.SemaphoreType.DMA,
    ],
)


def _lin1_body(x_ref, w_ref, deg_ref, g_ref, dis_ref):
    deg = deg_ref[0] + deg_ref[1] + 1.0
    dis = lax.rsqrt(deg)
    h = jnp.dot(x_ref[...], w_ref[...], preferred_element_type=jnp.float32)
    g_ref[pl.ds(0, N), :] = h * dis[:N]
    g_ref[pl.ds(N, N_PAD - N), :] = jnp.zeros((N_PAD - N, D), jnp.float32)
    dis_ref[...] = dis


_lin1 = pl.pallas_call(
    _lin1_body,
    out_shape=(jax.ShapeDtypeStruct((N_PAD, D), jnp.float32),
               jax.ShapeDtypeStruct((N_PAD, 1), jnp.float32)),
)


def _lin2_body(a_ref, g_ref, dis_ref, b_ref, w_ref, g2_ref):
    s = a_ref[0] + a_ref[1] + g_ref[...]
    z = jnp.maximum(s * dis_ref[...] + b_ref[...], 0.0)
    g2_ref[...] = jnp.dot(z, w_ref[...],
                          preferred_element_type=jnp.float32) * dis_ref[...]


_lin2 = pl.pallas_call(
    _lin2_body,
    out_shape=jax.ShapeDtypeStruct((N_PAD, D), jnp.float32),
)


def _out_body(a_ref, g_ref, dis_ref, b_ref, o_ref):
    o_ref[...] = (a_ref[0] + a_ref[1] + g_ref[...]) * dis_ref[...] + b_ref[...]


_out = pl.pallas_call(
    _out_body,
    out_shape=jax.ShapeDtypeStruct((N_PAD, D), jnp.float32),
)


def kernel(x, edge_index, W1, b1, W2, b2):
    pad = DUMMY + (jnp.arange(E_PAD - E, dtype=jnp.int32) % (N_PAD - N - 1))
    src = jnp.concatenate([edge_index[0], pad]).reshape(NW, CPW, K)
    dst = jnp.concatenate([edge_index[1], pad]).reshape(NW, CPW, K)

    deg2 = _deg(dst).reshape(NC, N_PAD, 1)
    g1, dis = _lin1(x, W1, deg2)
    A1 = _agg(g1, src, dst)
    g2 = _lin2(A1, g1, dis, b1.reshape(1, D), W2)
    A2 = _agg(g2, src, dst)
    out = _out(A2, g2, dis, b2.reshape(1, D))
    return out[:N]


# speedup vs baseline: 33.3585x; 1.0159x over previous
import functools

import jax
import jax.numpy as jnp
from jax import lax
from jax.experimental import pallas as pl
from jax.experimental.pallas import tpu as pltpu
from jax.experimental.pallas import tpu_sc as plsc

N = 10000
E = 320000
D = 128

NC = 2
NS = 16
NW = NC * NS
L = 16

N_PAD = 10240
SLAB = N_PAD // NS
K = 128
CPW = 80
EW = CPW * K
E_PAD = EW * NW
DUMMY = N

_MESH = plsc.VectorSubcoreMesh(core_axis_name="c", subcore_axis_name="s")


def _deg_body(e_hbm, out_hbm, dacc, idx_d, ones_v, zrow):
    cid = lax.axis_index("c")
    sid = lax.axis_index("s")
    wid = sid * NC + cid

    def _zfill(i, carry):
        zrow[pl.ds(i * L, L)] = jnp.zeros((L,), jnp.float32)
        return carry

    lax.fori_loop(0, SLAB // L, _zfill, 0)
    for j in range(K // L):
        ones_v[pl.ds(j * L, L)] = jnp.ones((L,), jnp.float32)

    pltpu.sync_copy(zrow, dacc.at[pl.ds(sid * SLAB, SLAB)])
    pltpu.sync_copy(e_hbm.at[1, wid], idx_d)
    plsc.subcore_barrier()

    def _chunk(c, carry):
        pltpu.sync_copy(ones_v, dacc.at[idx_d.at[c]], add=True)
        return carry

    lax.fori_loop(0, CPW, _chunk, 0)
    plsc.subcore_barrier()
    pltpu.sync_copy(dacc.at[pl.ds(sid * SLAB, SLAB)],
                    out_hbm.at[cid, pl.ds(sid * SLAB, SLAB)])


_deg = pl.kernel(
    _deg_body,
    out_type=jax.ShapeDtypeStruct((NC, N_PAD), jnp.float32),
    mesh=_MESH,
    scratch_types=[
        pltpu.VMEM_SHARED((N_PAD,), jnp.float32),
        pltpu.VMEM((CPW, K), jnp.int32),
        pltpu.VMEM((K,), jnp.float32),
        pltpu.VMEM((SLAB,), jnp.float32),
    ],
)


SB = 8
NSB = CPW // SB


def _agg_body(g_hbm, e_hbm, out_hbm,
              acc, idx_s, idx_d, rows, zbuf, sem0, sem1, semi0, semi1):
    cid = lax.axis_index("c")
    sid = lax.axis_index("s")
    wid = sid * NC + cid

    sems = (sem0, sem1)
    semi = (semi0, semi1)

    def _start_idx(sb, b):
        pltpu.async_copy(e_hbm.at[0, wid, pl.ds(sb * SB, SB)], idx_s.at[b],
                         semi[b])
        pltpu.async_copy(e_hbm.at[1, wid, pl.ds(sb * SB, SB)], idx_d.at[b],
                         semi[b])

    def _wait_idx(sb, b):
        pltpu.make_async_copy(e_hbm.at[0, wid, pl.ds(sb * SB, SB)],
                              idx_s.at[b], semi[b]).wait()
        pltpu.make_async_copy(e_hbm.at[1, wid, pl.ds(sb * SB, SB)],
                              idx_d.at[b], semi[b]).wait()

    _start_idx(0, 0)

    def _zfill(i, carry):
        for j in range(D // L):
            zbuf[i, pl.ds(j * L, L)] = jnp.zeros((L,), jnp.float32)
        return carry

    lax.fori_loop(0, 8, _zfill, 0)

    def _zslab(c, carry):
        pltpu.sync_copy(zbuf, acc.at[pl.ds(sid * SLAB + c * 8, 8)])
        return carry

    lax.fori_loop(0, SLAB // 8, _zslab, 0)

    _wait_idx(0, 0)
    pltpu.async_copy(g_hbm.at[idx_s.at[0, 0]], rows.at[0], sem0)
    plsc.subcore_barrier()

    def _pair(pair, carry):
        for p in range(2):
            sb = pair * 2 + p

            @pl.when(sb + 1 < NSB)
            def _prefetch():
                _start_idx(sb + 1, 1 - p)

            def _chunks(i, carry2):
                for b in range(2):
                    c = i * 2 + b
                    pltpu.async_copy(g_hbm.at[idx_s.at[p, c + 1]],
                                     rows.at[1 - b], sems[1 - b])
                    pltpu.make_async_copy(g_hbm.at[idx_s.at[p, c]],
                                          rows.at[b], sems[b]).wait()
                    pltpu.sync_copy(rows.at[b], acc.at[idx_d.at[p, c]],
                                    add=True)
                return carry2

            lax.fori_loop(0, SB // 2 - 1, _chunks, 0)

            pltpu.async_copy(g_hbm.at[idx_s.at[p, SB - 1]], rows.at[1],
                             sems[1])
            pltpu.make_async_copy(g_hbm.at[idx_s.at[p, SB - 2]], rows.at[0],
                                  sems[0]).wait()
            pltpu.sync_copy(rows.at[0], acc.at[idx_d.at[p, SB - 2]], add=True)

            @pl.when(sb + 1 < NSB)
            def _prime_next():
                _wait_idx(sb + 1, 1 - p)
                pltpu.async_copy(g_hbm.at[idx_s.at[1 - p, 0]], rows.at[0],
                                 sems[0])

            pltpu.make_async_copy(g_hbm.at[idx_s.at[p, SB - 1]], rows.at[1],
                                  sems[1]).wait()
            pltpu.sync_copy(rows.at[1], acc.at[idx_d.at[p, SB - 1]], add=True)
        return carry

    lax.fori_loop(0, NSB // 2, _pair, 0)
    plsc.subcore_barrier()
    pltpu.sync_copy(acc.at[pl.ds(sid * SLAB, SLAB)],
                    out_hbm.at[cid, pl.ds(sid * SLAB, SLAB)])


_agg = pl.kernel(
    _agg_body,
    out_type=jax.ShapeDtypeStruct((NC, N_PAD, D), jnp.float32),
    mesh=_MESH,
    scratch_types=[
        pltpu.VMEM_SHARED((N_PAD, D), jnp.float32),
        pltpu.VMEM((2, SB, K), jnp.int32),
        pltpu.VMEM((2, SB, K), jnp.int32),
        pltpu.VMEM((2, K, D), jnp.float32),
        pltpu.VMEM((8, D), jnp.float32),
        pltpu.SemaphoreType.DMA,
        pltpu.SemaphoreType.DMA,
        pltpu.SemaphoreType.DMA,
        pltpu.SemaphoreType.DMA,
    ],
)


def _lin1_body(x_ref, w_ref, deg_ref, g_ref, dis_ref):
    deg = deg_ref[0] + deg_ref[1] + 1.0
    dis = lax.rsqrt(deg)
    h = jnp.dot(x_ref[...], w_ref[...], preferred_element_type=jnp.float32)
    g_ref[pl.ds(0, N), :] = h * dis[:N]
    g_ref[pl.ds(N, N_PAD - N), :] = jnp.zeros((N_PAD - N, D), jnp.float32)
    dis_ref[...] = dis


_lin1 = pl.pallas_call(
    _lin1_body,
    out_shape=(jax.ShapeDtypeStruct((N_PAD, D), jnp.float32),
               jax.ShapeDtypeStruct((N_PAD, 1), jnp.float32)),
)


def _lin2_body(a_ref, g_ref, dis_ref, b_ref, w_ref, g2_ref):
    s = a_ref[0] + a_ref[1] + g_ref[...]
    z = jnp.maximum(s * dis_ref[...] + b_ref[...], 0.0)
    g2_ref[...] = jnp.dot(z, w_ref[...],
                          preferred_element_type=jnp.float32) * dis_ref[...]


_lin2 = pl.pallas_call(
    _lin2_body,
    out_shape=jax.ShapeDtypeStruct((N_PAD, D), jnp.float32),
)


def _out_body(a_ref, g_ref, dis_ref, b_ref, o_ref):
    s = (a_ref[0, pl.ds(0, N), :] + a_ref[1, pl.ds(0, N), :]
         + g_ref[pl.ds(0, N), :])
    o_ref[...] = s * dis_ref[pl.ds(0, N), :] + b_ref[...]


_out = pl.pallas_call(
    _out_body,
    out_shape=jax.ShapeDtypeStruct((N, D), jnp.float32),
)


def kernel(x, edge_index, W1, b1, W2, b2):
    pad = DUMMY + (jnp.arange(E_PAD - E, dtype=jnp.int32) % (N_PAD - N - 1))
    edges = jnp.concatenate(
        [edge_index, jnp.broadcast_to(pad, (2, E_PAD - E))], axis=1
    ).reshape(2, NW, CPW, K)

    deg2 = _deg(edges).reshape(NC, N_PAD, 1)
    g1, dis = _lin1(x, W1, deg2)
    A1 = _agg(g1, edges)
    g2 = _lin2(A1, g1, dis, b1.reshape(1, D), W2)
    A2 = _agg(g2, edges)
    return _out(A2, g2, dis, b2.reshape(1, D))


# speedup vs baseline: 35.4665x; 1.0632x over previous
import functools

import jax
import jax.numpy as jnp
from jax import lax
from jax.experimental import pallas as pl
from jax.experimental.pallas import tpu as pltpu
from jax.experimental.pallas import tpu_sc as plsc

N = 10000
E = 320000
D = 128

NC = 2
NS = 16
NW = NC * NS
L = 16

N_PAD = 10240
SLAB = N_PAD // NS
K = 128
CPW = 80
EW = CPW * K
E_PAD = EW * NW
DUMMY = N

_MESH = plsc.VectorSubcoreMesh(core_axis_name="c", subcore_axis_name="s")


def _deg_body(e_hbm, out_hbm, dacc, idx_d, ones_v, zrow):
    cid = lax.axis_index("c")
    sid = lax.axis_index("s")
    wid = sid * NC + cid

    def _zfill(i, carry):
        zrow[pl.ds(i * L, L)] = jnp.zeros((L,), jnp.float32)
        return carry

    lax.fori_loop(0, SLAB // L, _zfill, 0)
    for j in range(K // L):
        ones_v[pl.ds(j * L, L)] = jnp.ones((L,), jnp.float32)

    pltpu.sync_copy(zrow, dacc.at[pl.ds(sid * SLAB, SLAB)])
    pltpu.sync_copy(e_hbm.at[1, wid], idx_d)
    plsc.subcore_barrier()

    def _chunk(c, carry):
        pltpu.sync_copy(ones_v, dacc.at[idx_d.at[c]], add=True)
        return carry

    lax.fori_loop(0, CPW, _chunk, 0)
    plsc.subcore_barrier()
    pltpu.sync_copy(dacc.at[pl.ds(sid * SLAB, SLAB)],
                    out_hbm.at[cid, pl.ds(sid * SLAB, SLAB)])


_deg = pl.kernel(
    _deg_body,
    out_type=jax.ShapeDtypeStruct((NC, N_PAD), jnp.float32),
    mesh=_MESH,
    scratch_types=[
        pltpu.VMEM_SHARED((N_PAD,), jnp.float32),
        pltpu.VMEM((CPW, K), jnp.int32),
        pltpu.VMEM((K,), jnp.float32),
        pltpu.VMEM((SLAB,), jnp.float32),
    ],
)


SB = 8
NSB = CPW // SB


def _agg_body(g_hbm, e_hbm, out_hbm,
              acc, idx_s, idx_d, rows, zbuf, sem0, sem1, semi0, semi1, semz):
    cid = lax.axis_index("c")
    sid = lax.axis_index("s")
    wid = sid * NC + cid

    sems = (sem0, sem1)
    semi = (semi0, semi1)

    def _start_idx(sb, b):
        pltpu.async_copy(e_hbm.at[0, wid, pl.ds(sb * SB, SB)], idx_s.at[b],
                         semi[b])
        pltpu.async_copy(e_hbm.at[1, wid, pl.ds(sb * SB, SB)], idx_d.at[b],
                         semi[b])

    def _wait_idx(sb, b):
        pltpu.make_async_copy(e_hbm.at[0, wid, pl.ds(sb * SB, SB)],
                              idx_s.at[b], semi[b]).wait()
        pltpu.make_async_copy(e_hbm.at[1, wid, pl.ds(sb * SB, SB)],
                              idx_d.at[b], semi[b]).wait()

    _start_idx(0, 0)

    def _zfill(i, carry):
        for j in range(D // L):
            zbuf[i, pl.ds(j * L, L)] = jnp.zeros((L,), jnp.float32)
        return carry

    lax.fori_loop(0, 64, _zfill, 0)

    def _zslab(c, carry):
        pltpu.async_copy(zbuf, acc.at[pl.ds(sid * SLAB + c * 64, 64)], semz)
        return carry

    lax.fori_loop(0, SLAB // 64, _zslab, 0)

    _wait_idx(0, 0)
    pltpu.async_copy(g_hbm.at[idx_s.at[0, 0]], rows.at[0], sem0)

    def _zdrain(c, carry):
        pltpu.make_async_copy(zbuf, acc.at[pl.ds(sid * SLAB + c * 64, 64)],
                              semz).wait()
        return carry

    lax.fori_loop(0, SLAB // 64, _zdrain, 0)
    plsc.subcore_barrier()

    def _pair(pair, carry):
        for p in range(2):
            sb = pair * 2 + p

            @pl.when(sb + 1 < NSB)
            def _prefetch():
                _start_idx(sb + 1, 1 - p)

            def _chunks(i, carry2):
                for b in range(2):
                    c = i * 2 + b
                    pltpu.async_copy(g_hbm.at[idx_s.at[p, c + 1]],
                                     rows.at[1 - b], sems[1 - b])
                    pltpu.make_async_copy(g_hbm.at[idx_s.at[p, c]],
                                          rows.at[b], sems[b]).wait()
                    pltpu.sync_copy(rows.at[b], acc.at[idx_d.at[p, c]],
                                    add=True)
                return carry2

            lax.fori_loop(0, SB // 2 - 1, _chunks, 0)

            pltpu.async_copy(g_hbm.at[idx_s.at[p, SB - 1]], rows.at[1],
                             sems[1])
            pltpu.make_async_copy(g_hbm.at[idx_s.at[p, SB - 2]], rows.at[0],
                                  sems[0]).wait()
            pltpu.sync_copy(rows.at[0], acc.at[idx_d.at[p, SB - 2]], add=True)

            @pl.when(sb + 1 < NSB)
            def _prime_next():
                _wait_idx(sb + 1, 1 - p)
                pltpu.async_copy(g_hbm.at[idx_s.at[1 - p, 0]], rows.at[0],
                                 sems[0])

            pltpu.make_async_copy(g_hbm.at[idx_s.at[p, SB - 1]], rows.at[1],
                                  sems[1]).wait()
            pltpu.sync_copy(rows.at[1], acc.at[idx_d.at[p, SB - 1]], add=True)
        return carry

    lax.fori_loop(0, NSB // 2, _pair, 0)
    plsc.subcore_barrier()
    pltpu.sync_copy(acc.at[pl.ds(sid * SLAB, SLAB)],
                    out_hbm.at[cid, pl.ds(sid * SLAB, SLAB)])


_agg = pl.kernel(
    _agg_body,
    out_type=jax.ShapeDtypeStruct((NC, N_PAD, D), jnp.float32),
    mesh=_MESH,
    scratch_types=[
        pltpu.VMEM_SHARED((N_PAD, D), jnp.float32),
        pltpu.VMEM((2, SB, K), jnp.int32),
        pltpu.VMEM((2, SB, K), jnp.int32),
        pltpu.VMEM((2, K, D), jnp.float32),
        pltpu.VMEM((64, D), jnp.float32),
        pltpu.SemaphoreType.DMA,
        pltpu.SemaphoreType.DMA,
        pltpu.SemaphoreType.DMA,
        pltpu.SemaphoreType.DMA,
        pltpu.SemaphoreType.DMA,
    ],
)


def _lin1_body(x_ref, w_ref, deg_ref, g_ref, dis_ref):
    deg = deg_ref[0] + deg_ref[1] + 1.0
    dis = lax.rsqrt(deg)
    h = jnp.dot(x_ref[...], w_ref[...], preferred_element_type=jnp.float32)
    g_ref[pl.ds(0, N), :] = h * dis[:N]
    g_ref[pl.ds(N, N_PAD - N), :] = jnp.zeros((N_PAD - N, D), jnp.float32)
    dis_ref[...] = dis


_lin1 = pl.pallas_call(
    _lin1_body,
    out_shape=(jax.ShapeDtypeStruct((N_PAD, D), jnp.float32),
               jax.ShapeDtypeStruct((N_PAD, 1), jnp.float32)),
)


def _lin2_body(a_ref, g_ref, dis_ref, b_ref, w_ref, g2_ref):
    s = a_ref[0] + a_ref[1] + g_ref[...]
    z = jnp.maximum(s * dis_ref[...] + b_ref[...], 0.0)
    g2_ref[...] = jnp.dot(z, w_ref[...],
                          preferred_element_type=jnp.float32) * dis_ref[...]


_lin2 = pl.pallas_call(
    _lin2_body,
    out_shape=jax.ShapeDtypeStruct((N_PAD, D), jnp.float32),
)


def _out_body(a_ref, g_ref, dis_ref, b_ref, o_ref):
    s = (a_ref[0, pl.ds(0, N), :] + a_ref[1, pl.ds(0, N), :]
         + g_ref[pl.ds(0, N), :])
    o_ref[...] = s * dis_ref[pl.ds(0, N), :] + b_ref[...]


_out = pl.pallas_call(
    _out_body,
    out_shape=jax.ShapeDtypeStruct((N, D), jnp.float32),
)


def kernel(x, edge_index, W1, b1, W2, b2):
    pad = DUMMY + (jnp.arange(E_PAD - E, dtype=jnp.int32) % (N_PAD - N - 1))
    edges = jnp.concatenate(
        [edge_index, jnp.broadcast_to(pad, (2, E_PAD - E))], axis=1
    ).reshape(2, NW, CPW, K)

    deg2 = _deg(edges).reshape(NC, N_PAD, 1)
    g1, dis = _lin1(x, W1, deg2)
    A1 = _agg(g1, edges)
    g2 = _lin2(A1, g1, dis, b1.reshape(1, D), W2)
    A2 = _agg(g2, edges)
    return _out(A2, g2, dis, b2.reshape(1, D))


# speedup vs baseline: 37.1112x; 1.0464x over previous
import functools

import jax
import jax.numpy as jnp
from jax import lax
from jax.experimental import pallas as pl
from jax.experimental.pallas import tpu as pltpu
from jax.experimental.pallas import tpu_sc as plsc

N = 10000
E = 320000
D = 128

NC = 2
NS = 16
NW = NC * NS
L = 16

N_PAD = 10240
SLAB = N_PAD // NS
K = 128
CPW = 80
EW = CPW * K
E_PAD = EW * NW
DUMMY = N

_MESH = plsc.VectorSubcoreMesh(core_axis_name="c", subcore_axis_name="s")


def _deg_body(e_hbm, out_hbm, dacc, idx_d, ones_v, zrow):
    cid = lax.axis_index("c")
    sid = lax.axis_index("s")
    wid = sid * NC + cid

    def _zfill(i, carry):
        zrow[pl.ds(i * L, L)] = jnp.zeros((L,), jnp.float32)
        return carry

    lax.fori_loop(0, SLAB // L, _zfill, 0)
    for j in range(K // L):
        ones_v[pl.ds(j * L, L)] = jnp.ones((L,), jnp.float32)

    pltpu.sync_copy(zrow, dacc.at[pl.ds(sid * SLAB, SLAB)])
    pltpu.sync_copy(e_hbm.at[1, wid], idx_d)
    plsc.subcore_barrier()

    def _chunk(c, carry):
        pltpu.sync_copy(ones_v, dacc.at[idx_d.at[c]], add=True)
        return carry

    lax.fori_loop(0, CPW, _chunk, 0)
    plsc.subcore_barrier()
    pltpu.sync_copy(dacc.at[pl.ds(sid * SLAB, SLAB)],
                    out_hbm.at[cid, pl.ds(sid * SLAB, SLAB)])


_deg = pl.kernel(
    _deg_body,
    out_type=jax.ShapeDtypeStruct((NC, N_PAD), jnp.float32),
    mesh=_MESH,
    scratch_types=[
        pltpu.VMEM_SHARED((N_PAD,), jnp.float32),
        pltpu.VMEM((CPW, K), jnp.int32),
        pltpu.VMEM((K,), jnp.float32),
        pltpu.VMEM((SLAB,), jnp.float32),
    ],
)


SB = 8
NSB = CPW // SB


def _agg_body(g_hbm, e_hbm, out_hbm,
              acc, idx_s, idx_d, rows, zbuf, sem0, sem1, semi0, semi1, semz):
    cid = lax.axis_index("c")
    sid = lax.axis_index("s")
    wid = sid * NC + cid

    sems = (sem0, sem1)
    semi = (semi0, semi1)

    def _start_idx(sb, b):
        pltpu.async_copy(e_hbm.at[0, wid, pl.ds(sb * SB, SB)], idx_s.at[b],
                         semi[b])
        pltpu.async_copy(e_hbm.at[1, wid, pl.ds(sb * SB, SB)], idx_d.at[b],
                         semi[b])

    def _wait_idx(sb, b):
        pltpu.make_async_copy(e_hbm.at[0, wid, pl.ds(sb * SB, SB)],
                              idx_s.at[b], semi[b]).wait()
        pltpu.make_async_copy(e_hbm.at[1, wid, pl.ds(sb * SB, SB)],
                              idx_d.at[b], semi[b]).wait()

    _start_idx(0, 0)

    def _zfill(i, carry):
        for j in range(D // L):
            zbuf[i, pl.ds(j * L, L)] = jnp.zeros((L,), jnp.float32)
        return carry

    lax.fori_loop(0, 64, _zfill, 0)

    def _zslab(c, carry):
        pltpu.async_copy(zbuf, acc.at[pl.ds(sid * SLAB + c * 64, 64)], semz)
        return carry

    lax.fori_loop(0, SLAB // 64, _zslab, 0)

    _wait_idx(0, 0)
    pltpu.async_copy(g_hbm.at[idx_s.at[0, 0]], rows.at[0], sem0)

    def _zdrain(c, carry):
        pltpu.make_async_copy(zbuf, acc.at[pl.ds(sid * SLAB + c * 64, 64)],
                              semz).wait()
        return carry

    lax.fori_loop(0, SLAB // 64, _zdrain, 0)
    plsc.subcore_barrier()

    def _pair(pair, carry):
        for p in range(2):
            sb = pair * 2 + p

            @pl.when(sb + 1 < NSB)
            def _prefetch():
                _start_idx(sb + 1, 1 - p)

            def _chunks(i, carry2):
                for b in range(2):
                    c = i * 2 + b
                    pltpu.async_copy(g_hbm.at[idx_s.at[p, c + 1]],
                                     rows.at[1 - b], sems[1 - b])
                    pltpu.make_async_copy(g_hbm.at[idx_s.at[p, c]],
                                          rows.at[b], sems[b]).wait()
                    pltpu.sync_copy(rows.at[b], acc.at[idx_d.at[p, c]],
                                    add=True)
                return carry2

            lax.fori_loop(0, SB // 2 - 1, _chunks, 0)

            pltpu.async_copy(g_hbm.at[idx_s.at[p, SB - 1]], rows.at[1],
                             sems[1])
            pltpu.make_async_copy(g_hbm.at[idx_s.at[p, SB - 2]], rows.at[0],
                                  sems[0]).wait()
            pltpu.sync_copy(rows.at[0], acc.at[idx_d.at[p, SB - 2]], add=True)

            @pl.when(sb + 1 < NSB)
            def _prime_next():
                _wait_idx(sb + 1, 1 - p)
                pltpu.async_copy(g_hbm.at[idx_s.at[1 - p, 0]], rows.at[0],
                                 sems[0])

            pltpu.make_async_copy(g_hbm.at[idx_s.at[p, SB - 1]], rows.at[1],
                                  sems[1]).wait()
            pltpu.sync_copy(rows.at[1], acc.at[idx_d.at[p, SB - 1]], add=True)
        return carry

    lax.fori_loop(0, NSB // 2, _pair, 0)
    plsc.subcore_barrier()
    pltpu.sync_copy(acc.at[pl.ds(sid * SLAB, SLAB)],
                    out_hbm.at[cid, pl.ds(sid * SLAB, SLAB)])


_agg = pl.kernel(
    _agg_body,
    out_type=jax.ShapeDtypeStruct((NC, N_PAD, D), jnp.float32),
    mesh=_MESH,
    scratch_types=[
        pltpu.VMEM_SHARED((N_PAD, D), jnp.float32),
        pltpu.VMEM((2, SB, K), jnp.int32),
        pltpu.VMEM((2, SB, K), jnp.int32),
        pltpu.VMEM((2, K, D), jnp.float32),
        pltpu.VMEM((64, D), jnp.float32),
        pltpu.SemaphoreType.DMA,
        pltpu.SemaphoreType.DMA,
        pltpu.SemaphoreType.DMA,
        pltpu.SemaphoreType.DMA,
        pltpu.SemaphoreType.DMA,
    ],
)


def _dis_col(deg_ref):
    dis80 = lax.rsqrt(deg_ref[0] + deg_ref[1] + 1.0)
    eye = (lax.broadcasted_iota(jnp.int32, (D, D), 0) ==
           lax.broadcasted_iota(jnp.int32, (D, D), 1)).astype(jnp.float32)
    dis3 = jnp.sum(dis80[:, None, :] * eye[None], axis=-1, keepdims=True)
    return jnp.reshape(dis3, (N_PAD, 1))


def _lin1_body(x_ref, w_ref, deg_ref, g_ref):
    dis = _dis_col(deg_ref)
    h = jnp.dot(x_ref[...], w_ref[...], preferred_element_type=jnp.float32)
    g_ref[pl.ds(0, N), :] = h * dis[:N]
    g_ref[pl.ds(N, N_PAD - N), :] = jnp.zeros((N_PAD - N, D), jnp.float32)


_lin1 = pl.pallas_call(
    _lin1_body,
    out_shape=jax.ShapeDtypeStruct((N_PAD, D), jnp.float32),
)


def _lin2_body(a_ref, g_ref, deg_ref, b_ref, w_ref, g2_ref):
    dis = _dis_col(deg_ref)
    s = a_ref[0] + a_ref[1] + g_ref[...]
    z = jnp.maximum(s * dis + b_ref[...], 0.0)
    g2_ref[...] = jnp.dot(z, w_ref[...],
                          preferred_element_type=jnp.float32) * dis


_lin2 = pl.pallas_call(
    _lin2_body,
    out_shape=jax.ShapeDtypeStruct((N_PAD, D), jnp.float32),
)


def _out_body(a_ref, g_ref, deg_ref, b_ref, o_ref):
    dis = _dis_col(deg_ref)
    s = (a_ref[0, pl.ds(0, N), :] + a_ref[1, pl.ds(0, N), :]
         + g_ref[pl.ds(0, N), :])
    o_ref[...] = s * dis[:N] + b_ref[...]


_out = pl.pallas_call(
    _out_body,
    out_shape=jax.ShapeDtypeStruct((N, D), jnp.float32),
)


def kernel(x, edge_index, W1, b1, W2, b2):
    pad = DUMMY + (jnp.arange(E_PAD - E, dtype=jnp.int32) % (N_PAD - N - 1))
    edges = jnp.concatenate(
        [edge_index, jnp.broadcast_to(pad, (2, E_PAD - E))], axis=1
    ).reshape(2, NW, CPW, K)

    degc = _deg(edges).reshape(NC, N_PAD // D, D)
    g1 = _lin1(x, W1, degc)
    A1 = _agg(g1, edges)
    g2 = _lin2(A1, g1, degc, b1.reshape(1, D), W2)
    A2 = _agg(g2, edges)
    return _out(A2, g2, degc, b2.reshape(1, D))
